# baseline (speedup 1.0000x reference)
import functools

import jax
import jax.numpy as jnp
from jax import lax
from jax.experimental import pallas as pl
from jax.experimental.pallas import tpu as pltpu
from jax.experimental.pallas import tpu_sc as plsc


_CHUNK = 128


@functools.lru_cache(maxsize=None)
def _build(B, V, D):
  info = plsc.get_sparse_core_info()
  NC, NS = info.num_cores, info.num_subcores
  NW = NC * NS
  b_per_w = B // NW
  n_chunks = b_per_w // _CHUNK
  mesh = plsc.VectorSubcoreMesh(core_axis_name="c", subcore_axis_name="s")

  @functools.partial(
      pl.kernel,
      mesh=mesh,
      out_type=jax.ShapeDtypeStruct((B, D), jnp.float32),
      scratch_types=[
          pltpu.VMEM((b_per_w,), jnp.int32),
          pltpu.VMEM((n_chunks, _CHUNK), jnp.int32),
          pltpu.VMEM((_CHUNK, 4 * D), jnp.float32),
          pltpu.VMEM((b_per_w, D), jnp.float32),
          pltpu.SemaphoreType.DMA,
      ],
      compiler_params=pltpu.CompilerParams(
          disable_bounds_checks=True, needs_layout_passes=False
      ),
  )
  def k(idx_hbm, t4_hbm, out_hbm, idx_v, sid_v, sup_v, rows_v, sem):
    wid = lax.axis_index("s") * NC + lax.axis_index("c")
    base = wid * b_per_w
    pltpu.sync_copy(idx_hbm.at[pl.ds(base, b_per_w)], idx_v)

    lanes = lax.iota(jnp.int32, 16)

    def build(g, _):
      v = idx_v[pl.ds(g * 16, 16)]
      p = g * 16 + lanes
      plsc.store_scatter(sid_v, [p >> 7, p & 127], v >> 2)
      return ()

    lax.fori_loop(0, b_per_w // 16, build, (), unroll=False)

    def chunk(c, _):
      pltpu.async_copy(t4_hbm.at[sid_v.at[c]], sup_v, sem).wait()

      def extract(g, _):
        r_vec = c * _CHUNK + g * 16 + lanes
        v = idx_v[pl.ds(c * _CHUNK + g * 16, 16)]
        col = (v & 3) * D
        for w in range(D):
          vals = plsc.load_gather(sup_v, [g * 16 + lanes, col + w])
          plsc.store_scatter(
              rows_v, [r_vec, jnp.full((16,), w, jnp.int32)], vals
          )
        return ()

      lax.fori_loop(0, _CHUNK // 16, extract, (), unroll=False)
      return ()

    lax.fori_loop(0, n_chunks, chunk, (), unroll=False)
    pltpu.sync_copy(rows_v, out_hbm.at[pl.ds(base, b_per_w)])

  return k


def kernel(severity_level, table):
  B = severity_level.shape[0]
  V, D = table.shape
  k = _build(B, V, D)
  t4 = table.reshape(V // 4, 4 * D)
  return k(severity_level.astype(jnp.int32), t4)


# speedup vs baseline: 1.7252x; 1.7252x over previous
import functools

import jax
import jax.numpy as jnp
from jax import lax
from jax.experimental import pallas as pl
from jax.experimental.pallas import tpu as pltpu
from jax.experimental.pallas import tpu_sc as plsc


@functools.lru_cache(maxsize=None)
def _build(B, V, D):
  info = plsc.get_sparse_core_info()
  NC, NS = info.num_cores, info.num_subcores
  NW = NC * NS
  b_per_w = B // NW
  mesh = plsc.VectorSubcoreMesh(core_axis_name="c", subcore_axis_name="s")

  @functools.partial(
      pl.kernel,
      mesh=mesh,
      out_type=jax.ShapeDtypeStruct((B, D), jnp.float32),
      scratch_types=[
          pltpu.VMEM((b_per_w,), jnp.int32),
          pltpu.VMEM((b_per_w, D), jnp.float32),
          pltpu.SemaphoreType.DMA,
      ],
      compiler_params=pltpu.CompilerParams(disable_bounds_checks=True),
  )
  def k(idx_hbm, table_hbm, out_hbm, idx_v, rows_v, sem):
    wid = lax.axis_index("s") * NC + lax.axis_index("c")
    base = wid * b_per_w
    pltpu.sync_copy(idx_hbm.at[pl.ds(base, b_per_w)], idx_v)

    @plsc.parallel_loop(0, b_per_w // 16, 1, unroll=2)
    def fire(g):
      v = idx_v[pl.ds(g * 16, 16)]
      for j in range(16):
        pltpu.async_copy(
            table_hbm.at[pl.ds(v[j], 1)],
            rows_v.at[pl.ds(g * 16 + j, 1)],
            sem,
        )
    pltpu.make_async_copy(table_hbm.at[pl.ds(0, b_per_w)], rows_v, sem).wait()
    pltpu.sync_copy(rows_v, out_hbm.at[pl.ds(base, b_per_w)])

  return k


def kernel(severity_level, table):
  B = severity_level.shape[0]
  V, D = table.shape
  k = _build(B, V, D)
  return k(severity_level.astype(jnp.int32), table)
